# Initial kernel scaffold; baseline (speedup 1.0000x reference)
import jax
import jax.numpy as jnp
from jax.experimental import pallas as pl


def kernel(x, W, b, gamma, beta, indices):
    raise NotImplementedError("write your pallas kernel here")


# speedup vs baseline: 1.0356x; 1.0356x over previous
import functools

import jax
import jax.numpy as jnp
from jax import lax
from jax.experimental import pallas as pl
from jax.experimental.pallas import tpu as pltpu
from jax.experimental.pallas import tpu_sc as plsc

N = 200000
NX = 512
NY = 512
IN_F = 6
OUT_F = 64
EPS = 1e-5

NW = 32
PPW = 12800
N_PAD = 16 * PPW
NVEC = PPW // 16
N_PASS = 32
BATCH = 128
SLOT = PPW + 16

STATS_BLK = 4096
H_BLK = 2048


def _stats_body(xt_ref, s1_ref, s2_ref, a1, a2):
    i = pl.program_id(0)

    @pl.when(i == 0)
    def _():
        a1[...] = jnp.zeros_like(a1)
        a2[...] = jnp.zeros_like(a2)

    xb = xt_ref[...]
    a1[...] += jnp.sum(xb, axis=1, keepdims=True)
    for k in range(IN_F):
        a2[:, k : k + 1] += jnp.sum(xb * xb[k : k + 1, :], axis=1, keepdims=True)

    @pl.when(i == pl.num_programs(0) - 1)
    def _():
        s1_ref[...] = a1[...]
        s2_ref[...] = a2[...]


def _stats(xt):
    return pl.pallas_call(
        _stats_body,
        grid=(N_PAD // STATS_BLK,),
        in_specs=[pl.BlockSpec((IN_F, STATS_BLK), lambda i: (0, i))],
        out_specs=[
            pl.BlockSpec((IN_F, 1), lambda i: (0, 0)),
            pl.BlockSpec((IN_F, IN_F), lambda i: (0, 0)),
        ],
        out_shape=[
            jax.ShapeDtypeStruct((IN_F, 1), jnp.float32),
            jax.ShapeDtypeStruct((IN_F, IN_F), jnp.float32),
        ],
        scratch_shapes=[
            pltpu.VMEM((IN_F, 1), jnp.float32),
            pltpu.VMEM((IN_F, IN_F), jnp.float32),
        ],
    )(xt)


def _h_body(x_ref, w_ref, b_ref, h_ref):
    acc = jnp.dot(x_ref[...], w_ref[...], preferred_element_type=jnp.float32)
    h_ref[...] = jnp.maximum(acc + b_ref[...], 0.0)


def _h_compute(x8, wf8, bf8):
    return pl.pallas_call(
        _h_body,
        grid=(N_PAD // H_BLK,),
        in_specs=[
            pl.BlockSpec((H_BLK, 8), lambda i: (i, 0)),
            pl.BlockSpec((8, 128), lambda i: (0, 0)),
            pl.BlockSpec((1, 128), lambda i: (0, 0)),
        ],
        out_specs=pl.BlockSpec((H_BLK, 128), lambda i: (i, 0)),
        out_shape=jax.ShapeDtypeStruct((N_PAD, 128), jnp.float32),
    )(x8, wf8, bf8)


D_RANGE = 4096
TRASH_ROW = D_RANGE
CHUNK = 3200


def _scatter_body(hmat, ixr, iyr, zsrc, grid_out, ixb, iyb, lists, ring,
                  rows, locb, pidb, zbuf, acc):
    c = lax.axis_index("c")
    s = lax.axis_index("s")
    base_pt = s * PPW
    lane = lax.iota(jnp.int32, 16)
    neg1 = jnp.full((16,), -1, jnp.int32)
    trash_pid = lane * 399

    pltpu.sync_copy(zsrc, zbuf)

    ns = (jnp.int32(0), jnp.int32(0))
    for ch in range(PPW // CHUNK):
        pltpu.sync_copy(ixr.at[pl.ds(base_pt + ch * CHUNK, CHUNK)], ixb)
        pltpu.sync_copy(iyr.at[pl.ds(base_pt + ch * CHUNK, CHUNK)], iyb)

        def pre_body(i, ns, ch=ch):
            ix = ixb[pl.ds(i * 16, 16)]
            iy = iyb[pl.ds(i * 16, 16)]
            cell = (ix << 9) + iy
            q = cell >> 12
            m = (q < 64) & ((q & 1) == c)
            e = (((q & 31) << 26) | ((cell & 4095) << 14)
                 | (ch * CHUNK + i * 16 + lane))
            g = q >> 5
            new = []
            for gg in range(2):
                mg = m & (g == gg)
                mi = jnp.where(mg, 1, 0)
                pos = (gg * SLOT + ns[gg] - 1) + plsc.cumsum(mi)
                plsc.store_scatter(lists, [pos], e, mask=mg)
                new.append(ns[gg] + jnp.sum(mi))
            return tuple(new)

        ns = lax.fori_loop(0, CHUNK // 16, pre_body, ns)
    for gg in range(2):
        lists[pl.ds(gg * SLOT + ns[gg], 16)] = neg1

    def fire(toff):
        for k in range(8):
            e = ring[pl.ds(toff + k * 16, 16)]
            pad = e < 0
            loc = (e >> 14) & 4095
            locb[0, pl.ds(k * 16, 16)] = jnp.where(pad, TRASH_ROW, loc)
            pidb[pl.ds(k * 16, 16)] = jnp.where(pad, trash_pid, e & 16383) + base_pt
        pltpu.sync_copy(hmat.at[pidb], rows)
        pltpu.sync_copy(rows, acc.at[locb.at[0]], add=True)

    my_row = s * (D_RANGE // 16)
    for k in range(4):
        pltpu.sync_copy(zbuf, acc.at[pl.ds(my_row + k * 64, 64)])
    plsc.subcore_barrier()

    for p in range(N_PASS):
        myq = 2 * p + c
        mrel = ((2 * p) & 31) + c
        g = p >> 4

        def scan_body(i, n, mrel=mrel, g=g):
            e = lists[pl.ds(g * SLOT + i * 16, 16)]
            m = (e >> 26) == mrel
            mi = jnp.where(m, 1, 0)
            pos = ((n - 1) + plsc.cumsum(mi)) & 255
            plsc.store_scatter(ring, [pos], e, mask=m)
            n2 = n + jnp.sum(mi)

            @pl.when((n2 >> 7) > (n >> 7))
            def _():
                fire(((n >> 7) << 7) & 255)

            return n2

        nvec_g = (ns[g] + 15) >> 4
        n = lax.fori_loop(0, nvec_g, scan_body, jnp.int32(0))

        @pl.when((n & 127) > 0)
        def _(n=n):
            for k in range(8):
                plsc.store_scatter(ring, [(n + k * 16 + lane) & 255], neg1)
            fire(((n >> 7) << 7) & 255)

        plsc.subcore_barrier()
        gbase = myq * D_RANGE + my_row
        pltpu.sync_copy(acc.at[pl.ds(my_row, D_RANGE // 16)],
                        grid_out.at[pl.ds(gbase, D_RANGE // 16)])
        if p < N_PASS - 1:
            for k in range(4):
                pltpu.sync_copy(zbuf, acc.at[pl.ds(my_row + k * 64, 64)])
        plsc.subcore_barrier()


def _make_scatter():
    return functools.partial(
        pl.kernel,
        out_type=jax.ShapeDtypeStruct((NX * NY, 128), jnp.float32),
        mesh=plsc.VectorSubcoreMesh(core_axis_name="c", subcore_axis_name="s"),
        compiler_params=pltpu.CompilerParams(needs_layout_passes=False),
        scratch_types=[
            pltpu.VMEM((CHUNK,), jnp.int32),
            pltpu.VMEM((CHUNK,), jnp.int32),
            pltpu.VMEM((2 * SLOT,), jnp.int32),
            pltpu.VMEM((256,), jnp.int32),
            pltpu.VMEM((BATCH, 128), jnp.float32),
            pltpu.VMEM((1, BATCH), jnp.int32),
            pltpu.VMEM((BATCH,), jnp.int32),
            pltpu.VMEM((64, 128), jnp.float32),
            pltpu.VMEM_SHARED((D_RANGE + 8, 128), jnp.float32),
        ],
    )(_scatter_body)


def kernel(x, W, b, gamma, beta, indices):
    x8 = jnp.zeros((N_PAD, 8), jnp.float32).at[:N, :IN_F].set(x)

    s1, s2 = _stats(x8[:, :IN_F].T)
    mean_x = s1[:, 0] / N
    e2 = s2 / N
    mh = W @ mean_x
    mean = mh + b
    eh2 = jnp.einsum("jk,kl,jl->j", W, e2, W) + 2.0 * b * mh + b * b
    var = jnp.maximum(eh2 - mean * mean, 0.0)
    sc = gamma * lax.rsqrt(var + EPS)
    wf8 = jnp.zeros((8, 128), jnp.float32).at[:IN_F, :OUT_F].set((W * sc[:, None]).T)
    bf8 = jnp.zeros((1, 128), jnp.float32).at[0, :OUT_F].set((b - mean) * sc + beta)

    h = _h_compute(x8, wf8, bf8)

    ix = jnp.full((N_PAD,), NX, jnp.int32).at[:N].set(indices[:, 0])
    iy = jnp.zeros((N_PAD,), jnp.int32).at[:N].set(indices[:, 1])
    zsrc = jnp.zeros((64, 128), jnp.float32)

    grid = _make_scatter()(h, ix, iy, zsrc)
    return grid[:, :OUT_F].reshape(NX, NY, OUT_F)


# speedup vs baseline: 1.0813x; 1.0441x over previous
import functools

import jax
import jax.numpy as jnp
from jax import lax
from jax.experimental import pallas as pl
from jax.experimental.pallas import tpu as pltpu
from jax.experimental.pallas import tpu_sc as plsc

N = 200000
NX = 512
NY = 512
IN_F = 6
OUT_F = 64
EPS = 1e-5

NW = 32
PPW = 12800
N_PAD = 16 * PPW
NVEC = PPW // 16
N_PASS = 32
BATCH = 128
ARENA = PPW + 32

STATS_BLK = 4096
H_BLK = 2048


def _stats_body(xt_ref, s1_ref, s2_ref, a1, a2):
    i = pl.program_id(0)

    @pl.when(i == 0)
    def _():
        a1[...] = jnp.zeros_like(a1)
        a2[...] = jnp.zeros_like(a2)

    xb = xt_ref[...]
    a1[...] += jnp.sum(xb, axis=1, keepdims=True)
    for k in range(IN_F):
        a2[:, k : k + 1] += jnp.sum(xb * xb[k : k + 1, :], axis=1, keepdims=True)

    @pl.when(i == pl.num_programs(0) - 1)
    def _():
        s1_ref[...] = a1[...]
        s2_ref[...] = a2[...]


def _stats(xt):
    return pl.pallas_call(
        _stats_body,
        grid=(N_PAD // STATS_BLK,),
        in_specs=[pl.BlockSpec((IN_F, STATS_BLK), lambda i: (0, i))],
        out_specs=[
            pl.BlockSpec((IN_F, 1), lambda i: (0, 0)),
            pl.BlockSpec((IN_F, IN_F), lambda i: (0, 0)),
        ],
        out_shape=[
            jax.ShapeDtypeStruct((IN_F, 1), jnp.float32),
            jax.ShapeDtypeStruct((IN_F, IN_F), jnp.float32),
        ],
        scratch_shapes=[
            pltpu.VMEM((IN_F, 1), jnp.float32),
            pltpu.VMEM((IN_F, IN_F), jnp.float32),
        ],
    )(xt)


def _h_body(x_ref, w_ref, b_ref, h_ref):
    acc = jnp.dot(x_ref[...], w_ref[...], preferred_element_type=jnp.float32)
    h_ref[...] = jnp.maximum(acc + b_ref[...], 0.0)


def _h_compute(x8, wf8, bf8):
    return pl.pallas_call(
        _h_body,
        grid=(N_PAD // H_BLK,),
        in_specs=[
            pl.BlockSpec((H_BLK, 8), lambda i: (i, 0)),
            pl.BlockSpec((8, 128), lambda i: (0, 0)),
            pl.BlockSpec((1, 128), lambda i: (0, 0)),
        ],
        out_specs=pl.BlockSpec((H_BLK, 128), lambda i: (i, 0)),
        out_shape=jax.ShapeDtypeStruct((N_PAD, 128), jnp.float32),
    )(x8, wf8, bf8)


D_RANGE = 4096
TRASH_ROW = D_RANGE
CHUNK = 3200


def _scatter_body(hmat, ixr, iyr, zsrc, grid_out, ixb, iyb, lists, ring,
                  rows0, rows1, locb, pidb0, pidb1, zbuf, acc, sem0, sem1):
    c = lax.axis_index("c")
    s = lax.axis_index("s")
    base_pt = s * PPW
    lane = lax.iota(jnp.int32, 16)
    neg1 = jnp.full((16,), -1, jnp.int32)
    trash_pid = lane * 399

    pltpu.sync_copy(zsrc, zbuf)

    ns = (jnp.int32(0), jnp.int32(0))
    for ch in range(PPW // CHUNK):
        pltpu.sync_copy(ixr.at[pl.ds(base_pt + ch * CHUNK, CHUNK)], ixb)
        pltpu.sync_copy(iyr.at[pl.ds(base_pt + ch * CHUNK, CHUNK)], iyb)

        def pre_body(i, ns, ch=ch):
            ix = ixb[pl.ds(i * 16, 16)]
            iy = iyb[pl.ds(i * 16, 16)]
            cell = (ix << 9) + iy
            q = cell >> 12
            m = (q < 64) & ((q & 1) == c)
            e = (((q & 31) << 26) | ((cell & 4095) << 14)
                 | (ch * CHUNK + i * 16 + lane))
            g = q >> 5
            csum0 = plsc.cumsum(jnp.where(m & (g == 0), 1, 0))
            csum1 = plsc.cumsum(jnp.where(m & (g == 1), 1, 0))
            plsc.store_scatter(lists, [(ns[0] - 1) + csum0], e, mask=m & (g == 0))
            plsc.store_scatter(lists, [(ARENA - ns[1]) - csum1], e, mask=m & (g == 1))
            return (ns[0] + csum0[15], ns[1] + csum1[15])

        ns = lax.fori_loop(0, CHUNK // 16, pre_body, ns)
    lists[pl.ds(ns[0], 16)] = neg1
    lists[pl.ds(ARENA - ns[1] - 16, 16)] = neg1

    def unpack(j):
        par = j & 1
        toff = par << 7
        for k in range(8):
            e = ring[pl.ds(toff + k * 16, 16)]
            pad = e < 0
            loc = (e >> 14) & 4095
            locb[par, pl.ds(k * 16, 16)] = jnp.where(pad, TRASH_ROW, loc)
            pid = jnp.where(pad, trash_pid, e & 16383) + base_pt

            @pl.when(par == 0)
            def _(pid=pid, k=k):
                pidb0[pl.ds(k * 16, 16)] = pid

            @pl.when(par == 1)
            def _(pid=pid, k=k):
                pidb1[pl.ds(k * 16, 16)] = pid

    def start_gather(j):
        @pl.when((j & 1) == 0)
        def _():
            pltpu.async_copy(hmat.at[pidb0], rows0, sem0)

        @pl.when((j & 1) == 1)
        def _():
            pltpu.async_copy(hmat.at[pidb1], rows1, sem1)

    def drain(j):
        @pl.when((j & 1) == 0)
        def _():
            pltpu.make_async_copy(hmat.at[pl.ds(0, BATCH)], rows0, sem0).wait()
            pltpu.sync_copy(rows0, acc.at[locb.at[0]], add=True)

        @pl.when((j & 1) == 1)
        def _():
            pltpu.make_async_copy(hmat.at[pl.ds(0, BATCH)], rows1, sem1).wait()
            pltpu.sync_copy(rows1, acc.at[locb.at[1]], add=True)

    def fire(j):
        @pl.when(j >= 1)
        def _():
            drain(j - 1)

        unpack(j)
        start_gather(j)

    my_row = s * (D_RANGE // 16)
    for k in range(4):
        pltpu.sync_copy(zbuf, acc.at[pl.ds(my_row + k * 64, 64)])
    plsc.subcore_barrier()

    for p in range(N_PASS):
        myq = 2 * p + c
        mrel = ((2 * p) & 31) + c
        g = p >> 4

        def scan_body(i, n, mrel=mrel, g=g):
            if g == 0:
                e = lists[pl.ds(i * 16, 16)]
            else:
                e = lists[pl.ds(ARENA - 16 - i * 16, 16)]
            m = (e >> 26) == mrel
            mi = jnp.where(m, 1, 0)
            pos = ((n - 1) + plsc.cumsum(mi)) & 255
            plsc.store_scatter(ring, [pos], e, mask=m)
            n2 = n + jnp.sum(mi)

            @pl.when((n2 >> 7) > (n >> 7))
            def _():
                fire(n >> 7)

            return n2

        nvec_g = (ns[g] + 15) >> 4
        n = lax.fori_loop(0, nvec_g, scan_body, jnp.int32(0))

        @pl.when((n & 127) > 0)
        def _(n=n):
            for k in range(8):
                plsc.store_scatter(ring, [(n + k * 16 + lane) & 255], neg1)
            fire(n >> 7)

        nb = (n + 127) >> 7

        @pl.when(nb >= 1)
        def _(nb=nb):
            drain(nb - 1)

        plsc.subcore_barrier()
        gbase = myq * D_RANGE + my_row
        pltpu.sync_copy(acc.at[pl.ds(my_row, D_RANGE // 16)],
                        grid_out.at[pl.ds(gbase, D_RANGE // 16)])
        if p < N_PASS - 1:
            for k in range(4):
                pltpu.sync_copy(zbuf, acc.at[pl.ds(my_row + k * 64, 64)])
        plsc.subcore_barrier()


def _make_scatter():
    return functools.partial(
        pl.kernel,
        out_type=jax.ShapeDtypeStruct((NX * NY, 128), jnp.float32),
        mesh=plsc.VectorSubcoreMesh(core_axis_name="c", subcore_axis_name="s"),
        compiler_params=pltpu.CompilerParams(needs_layout_passes=False),
        scratch_types=[
            pltpu.VMEM((CHUNK,), jnp.int32),
            pltpu.VMEM((CHUNK,), jnp.int32),
            pltpu.VMEM((ARENA,), jnp.int32),
            pltpu.VMEM((256,), jnp.int32),
            pltpu.VMEM((BATCH, 128), jnp.float32),
            pltpu.VMEM((BATCH, 128), jnp.float32),
            pltpu.VMEM((2, BATCH), jnp.int32),
            pltpu.VMEM((BATCH,), jnp.int32),
            pltpu.VMEM((BATCH,), jnp.int32),
            pltpu.VMEM((64, 128), jnp.float32),
            pltpu.VMEM_SHARED((D_RANGE + 8, 128), jnp.float32),
            pltpu.SemaphoreType.DMA,
            pltpu.SemaphoreType.DMA,
        ],
    )(_scatter_body)


def kernel(x, W, b, gamma, beta, indices):
    x8 = jnp.zeros((N_PAD, 8), jnp.float32).at[:N, :IN_F].set(x)

    s1, s2 = _stats(x8[:, :IN_F].T)
    mean_x = s1[:, 0] / N
    e2 = s2 / N
    mh = W @ mean_x
    mean = mh + b
    eh2 = jnp.einsum("jk,kl,jl->j", W, e2, W) + 2.0 * b * mh + b * b
    var = jnp.maximum(eh2 - mean * mean, 0.0)
    sc = gamma * lax.rsqrt(var + EPS)
    wf8 = jnp.zeros((8, 128), jnp.float32).at[:IN_F, :OUT_F].set((W * sc[:, None]).T)
    bf8 = jnp.zeros((1, 128), jnp.float32).at[0, :OUT_F].set((b - mean) * sc + beta)

    h = _h_compute(x8, wf8, bf8)

    ix = jnp.full((N_PAD,), NX, jnp.int32).at[:N].set(indices[:, 0])
    iy = jnp.zeros((N_PAD,), jnp.int32).at[:N].set(indices[:, 1])
    zsrc = jnp.zeros((64, 128), jnp.float32)

    grid = _make_scatter()(h, ix, iy, zsrc)
    return grid[:, :OUT_F].reshape(NX, NY, OUT_F)


# speedup vs baseline: 1.5009x; 1.3881x over previous
import functools

import jax
import jax.numpy as jnp
from jax import lax
from jax.experimental import pallas as pl
from jax.experimental.pallas import tpu as pltpu
from jax.experimental.pallas import tpu_sc as plsc

N = 200000
NX = 512
NY = 512
IN_F = 6
OUT_F = 64
EPS = 1e-5

NW = 32
PPW = 12800
N_PAD = 16 * PPW
NVEC = PPW // 16
N_PASS = 32
BATCH = 128
ARENA = PPW + 32

STATS_BLK = 4096
H_BLK = 2048


def _stats_body(xt_ref, s1_ref, s2_ref, a1, a2):
    i = pl.program_id(0)

    @pl.when(i == 0)
    def _():
        a1[...] = jnp.zeros_like(a1)
        a2[...] = jnp.zeros_like(a2)

    xb = xt_ref[...]
    a1[...] += jnp.sum(xb, axis=1, keepdims=True)
    for k in range(IN_F):
        a2[:, k : k + 1] += jnp.sum(xb * xb[k : k + 1, :], axis=1, keepdims=True)

    @pl.when(i == pl.num_programs(0) - 1)
    def _():
        s1_ref[...] = a1[...]
        s2_ref[...] = a2[...]


def _stats(xt):
    return pl.pallas_call(
        _stats_body,
        grid=(N_PAD // STATS_BLK,),
        in_specs=[pl.BlockSpec((8, STATS_BLK), lambda i: (0, i))],
        out_specs=[
            pl.BlockSpec((8, 1), lambda i: (0, 0)),
            pl.BlockSpec((8, 8), lambda i: (0, 0)),
        ],
        out_shape=[
            jax.ShapeDtypeStruct((8, 1), jnp.float32),
            jax.ShapeDtypeStruct((8, 8), jnp.float32),
        ],
        scratch_shapes=[
            pltpu.VMEM((8, 1), jnp.float32),
            pltpu.VMEM((8, 8), jnp.float32),
        ],
    )(xt)


def _h_body(x_ref, w_ref, b_ref, h_ref):
    acc = lax.dot_general(x_ref[...], w_ref[...], (((0,), (0,)), ((), ())),
                          preferred_element_type=jnp.float32)
    h_ref[...] = jnp.maximum(acc + b_ref[...], 0.0)


def _h_compute(x8, wf8, bf8):
    return pl.pallas_call(
        _h_body,
        grid=(N_PAD // H_BLK,),
        in_specs=[
            pl.BlockSpec((8, H_BLK), lambda i: (0, i)),
            pl.BlockSpec((8, 128), lambda i: (0, 0)),
            pl.BlockSpec((1, 128), lambda i: (0, 0)),
        ],
        out_specs=pl.BlockSpec((H_BLK, 128), lambda i: (i, 0)),
        out_shape=jax.ShapeDtypeStruct((N_PAD, 128), jnp.float32),
    )(x8, wf8, bf8)


D_RANGE = 4096
TRASH_ROW = D_RANGE
CHUNK = 3200


def _scatter_body(hmat, ixr, iyr, zsrc, grid_out, ixb, iyb, lists, ring,
                  rows0, rows1, locb, pidb0, pidb1, zbuf, acc, sem0, sem1):
    c = lax.axis_index("c")
    s = lax.axis_index("s")
    base_pt = s * PPW
    lane = lax.iota(jnp.int32, 16)
    neg1 = jnp.full((16,), -1, jnp.int32)
    trash_pid = lane * 399

    pltpu.sync_copy(zsrc, zbuf)

    ns = (jnp.int32(0), jnp.int32(0))
    for ch in range(PPW // CHUNK):
        pltpu.sync_copy(ixr.at[pl.ds(base_pt + ch * CHUNK, CHUNK)], ixb)
        pltpu.sync_copy(iyr.at[pl.ds(base_pt + ch * CHUNK, CHUNK)], iyb)

        def pre_body(i, ns, ch=ch):
            ix = ixb[pl.ds(i * 16, 16)]
            iy = iyb[pl.ds(i * 16, 16)]
            cell = (ix << 9) + iy
            q = cell >> 12
            m = (q < 64) & ((q & 1) == c)
            e = (((q & 31) << 26) | ((cell & 4095) << 14)
                 | (ch * CHUNK + i * 16 + lane))
            g = q >> 5
            csum0 = plsc.cumsum(jnp.where(m & (g == 0), 1, 0))
            csum1 = plsc.cumsum(jnp.where(m & (g == 1), 1, 0))
            plsc.store_scatter(lists, [(ns[0] - 1) + csum0], e, mask=m & (g == 0))
            plsc.store_scatter(lists, [(ARENA - ns[1]) - csum1], e, mask=m & (g == 1))
            return (ns[0] + csum0[15], ns[1] + csum1[15])

        ns = lax.fori_loop(0, CHUNK // 16, pre_body, ns)
    lists[pl.ds(ns[0], 16)] = neg1
    lists[pl.ds(ARENA - ns[1] - 16, 16)] = neg1

    def unpack(j):
        par = j & 1
        toff = par << 7
        for k in range(8):
            e = ring[pl.ds(toff + k * 16, 16)]
            pad = e < 0
            loc = (e >> 14) & 4095
            locb[par, pl.ds(k * 16, 16)] = jnp.where(pad, TRASH_ROW, loc)
            pid = jnp.where(pad, trash_pid, e & 16383) + base_pt

            @pl.when(par == 0)
            def _(pid=pid, k=k):
                pidb0[pl.ds(k * 16, 16)] = pid

            @pl.when(par == 1)
            def _(pid=pid, k=k):
                pidb1[pl.ds(k * 16, 16)] = pid

    def start_gather(j):
        @pl.when((j & 1) == 0)
        def _():
            pltpu.async_copy(hmat.at[pidb0], rows0, sem0)

        @pl.when((j & 1) == 1)
        def _():
            pltpu.async_copy(hmat.at[pidb1], rows1, sem1)

    def drain(j):
        @pl.when((j & 1) == 0)
        def _():
            pltpu.make_async_copy(hmat.at[pl.ds(0, BATCH)], rows0, sem0).wait()
            pltpu.sync_copy(rows0, acc.at[locb.at[0]], add=True)

        @pl.when((j & 1) == 1)
        def _():
            pltpu.make_async_copy(hmat.at[pl.ds(0, BATCH)], rows1, sem1).wait()
            pltpu.sync_copy(rows1, acc.at[locb.at[1]], add=True)

    def fire(j):
        @pl.when(j >= 1)
        def _():
            drain(j - 1)

        unpack(j)
        start_gather(j)

    my_row = s * (D_RANGE // 16)
    for k in range(4):
        pltpu.sync_copy(zbuf, acc.at[pl.ds(my_row + k * 64, 64)])
    plsc.subcore_barrier()

    for p in range(N_PASS):
        myq = 2 * p + c
        mrel = ((2 * p) & 31) + c
        g = p >> 4

        def scan_body(i, n, mrel=mrel, g=g):
            if g == 0:
                e = lists[pl.ds(i * 16, 16)]
            else:
                e = lists[pl.ds(ARENA - 16 - i * 16, 16)]
            m = (e >> 26) == mrel
            mi = jnp.where(m, 1, 0)
            pos = ((n - 1) + plsc.cumsum(mi)) & 255
            plsc.store_scatter(ring, [pos], e, mask=m)
            n2 = n + jnp.sum(mi)

            @pl.when((n2 >> 7) > (n >> 7))
            def _():
                fire(n >> 7)

            return n2

        nvec_g = (ns[g] + 15) >> 4
        n = lax.fori_loop(0, nvec_g, scan_body, jnp.int32(0))

        @pl.when((n & 127) > 0)
        def _(n=n):
            for k in range(8):
                plsc.store_scatter(ring, [(n + k * 16 + lane) & 255], neg1)
            fire(n >> 7)

        nb = (n + 127) >> 7

        @pl.when(nb >= 1)
        def _(nb=nb):
            drain(nb - 1)

        plsc.subcore_barrier()
        gbase = myq * D_RANGE + my_row
        pltpu.sync_copy(acc.at[pl.ds(my_row, D_RANGE // 16)],
                        grid_out.at[pl.ds(gbase, D_RANGE // 16)])
        if p < N_PASS - 1:
            for k in range(4):
                pltpu.sync_copy(zbuf, acc.at[pl.ds(my_row + k * 64, 64)])
        plsc.subcore_barrier()


def _make_scatter():
    return functools.partial(
        pl.kernel,
        out_type=jax.ShapeDtypeStruct((NX * NY, 128), jnp.float32),
        mesh=plsc.VectorSubcoreMesh(core_axis_name="c", subcore_axis_name="s"),
        compiler_params=pltpu.CompilerParams(needs_layout_passes=False),
        scratch_types=[
            pltpu.VMEM((CHUNK,), jnp.int32),
            pltpu.VMEM((CHUNK,), jnp.int32),
            pltpu.VMEM((ARENA,), jnp.int32),
            pltpu.VMEM((256,), jnp.int32),
            pltpu.VMEM((BATCH, 128), jnp.float32),
            pltpu.VMEM((BATCH, 128), jnp.float32),
            pltpu.VMEM((2, BATCH), jnp.int32),
            pltpu.VMEM((BATCH,), jnp.int32),
            pltpu.VMEM((BATCH,), jnp.int32),
            pltpu.VMEM((64, 128), jnp.float32),
            pltpu.VMEM_SHARED((D_RANGE + 8, 128), jnp.float32),
            pltpu.SemaphoreType.DMA,
            pltpu.SemaphoreType.DMA,
        ],
    )(_scatter_body)


def kernel(x, W, b, gamma, beta, indices):
    xt8 = jnp.zeros((8, N_PAD), jnp.float32).at[:IN_F, :N].set(x.T)

    s1, s2 = _stats(xt8)
    mean_x = s1[:IN_F, 0] / N
    e2 = s2[:IN_F, :IN_F] / N
    mh = W @ mean_x
    mean = mh + b
    eh2 = jnp.einsum("jk,kl,jl->j", W, e2, W) + 2.0 * b * mh + b * b
    var = jnp.maximum(eh2 - mean * mean, 0.0)
    sc = gamma * lax.rsqrt(var + EPS)
    wf8 = jnp.zeros((8, 128), jnp.float32).at[:IN_F, :OUT_F].set((W * sc[:, None]).T)
    bf8 = jnp.zeros((1, 128), jnp.float32).at[0, :OUT_F].set((b - mean) * sc + beta)

    h = _h_compute(xt8, wf8, bf8)

    ix = jnp.full((N_PAD,), NX, jnp.int32).at[:N].set(indices[:, 0])
    iy = jnp.zeros((N_PAD,), jnp.int32).at[:N].set(indices[:, 1])
    zsrc = jnp.zeros((64, 128), jnp.float32)

    grid = _make_scatter()(h, ix, iy, zsrc)
    return grid[:, :OUT_F].reshape(NX, NY, OUT_F)


# speedup vs baseline: 1.5118x; 1.0073x over previous
import functools

import jax
import jax.numpy as jnp
from jax import lax
from jax.experimental import pallas as pl
from jax.experimental.pallas import tpu as pltpu
from jax.experimental.pallas import tpu_sc as plsc

N = 200000
NX = 512
NY = 512
IN_F = 6
OUT_F = 64
EPS = 1e-5

NW = 32
PPW = 12800
N_PAD = 16 * PPW
NVEC = PPW // 16
N_PASS = 32
BATCH = 128
ARENA = PPW + 32

STATS_BLK = 4096
H_BLK = 2048


def _stats_body(xt_ref, s1_ref, s2_ref, a1, a2):
    i = pl.program_id(0)

    @pl.when(i == 0)
    def _():
        a1[...] = jnp.zeros_like(a1)
        a2[...] = jnp.zeros_like(a2)

    xb = xt_ref[...]
    a1[...] += jnp.sum(xb, axis=1, keepdims=True)
    for k in range(IN_F):
        a2[:, k : k + 1] += jnp.sum(xb * xb[k : k + 1, :], axis=1, keepdims=True)

    @pl.when(i == pl.num_programs(0) - 1)
    def _():
        s1_ref[...] = a1[...]
        s2_ref[...] = a2[...]


def _stats(xt):
    return pl.pallas_call(
        _stats_body,
        grid=(N_PAD // STATS_BLK,),
        in_specs=[pl.BlockSpec((8, STATS_BLK), lambda i: (0, i))],
        out_specs=[
            pl.BlockSpec((8, 1), lambda i: (0, 0)),
            pl.BlockSpec((8, 8), lambda i: (0, 0)),
        ],
        out_shape=[
            jax.ShapeDtypeStruct((8, 1), jnp.float32),
            jax.ShapeDtypeStruct((8, 8), jnp.float32),
        ],
        scratch_shapes=[
            pltpu.VMEM((8, 1), jnp.float32),
            pltpu.VMEM((8, 8), jnp.float32),
        ],
    )(xt)


def _h_body(x_ref, w_ref, b_ref, h_ref):
    acc = lax.dot_general(x_ref[...], w_ref[...], (((0,), (0,)), ((), ())),
                          preferred_element_type=jnp.float32)
    h_ref[...] = jnp.maximum(acc + b_ref[...], 0.0)


def _h_compute(x8, wf8, bf8):
    return pl.pallas_call(
        _h_body,
        grid=(N_PAD // H_BLK,),
        in_specs=[
            pl.BlockSpec((8, H_BLK), lambda i: (0, i)),
            pl.BlockSpec((8, 128), lambda i: (0, 0)),
            pl.BlockSpec((1, 128), lambda i: (0, 0)),
        ],
        out_specs=pl.BlockSpec((H_BLK, 128), lambda i: (i, 0)),
        out_shape=jax.ShapeDtypeStruct((N_PAD, 128), jnp.float32),
    )(x8, wf8, bf8)


D_RANGE = 4096
TRASH_ROW = D_RANGE
CHUNK = 3200


def _scatter_body(hmat, ixr, iyr, zsrc, grid_out, ixb, iyb, lists, ring,
                  rows0, rows1, locb, pidb0, pidb1, zbuf, acc, sem0, sem1):
    c = lax.axis_index("c")
    s = lax.axis_index("s")
    base_pt = s * PPW
    lane = lax.iota(jnp.int32, 16)
    neg1 = jnp.full((16,), -1, jnp.int32)
    trash_pid = lane * 399

    pltpu.sync_copy(zsrc, zbuf)

    ns = (jnp.int32(0), jnp.int32(0))
    for ch in range(PPW // CHUNK):
        pltpu.sync_copy(ixr.at[pl.ds(base_pt + ch * CHUNK, CHUNK)], ixb)
        pltpu.sync_copy(iyr.at[pl.ds(base_pt + ch * CHUNK, CHUNK)], iyb)

        def pre_body(i, ns, ch=ch):
            ix = ixb[pl.ds(i * 16, 16)]
            iy = iyb[pl.ds(i * 16, 16)]
            cell = (ix << 9) + iy
            q = cell >> 12
            m = (q < 64) & ((q & 1) == c)
            e = (((q & 31) << 26) | ((cell & 4095) << 14)
                 | (ch * CHUNK + i * 16 + lane))
            g = q >> 5
            csum0 = plsc.cumsum(jnp.where(m & (g == 0), 1, 0))
            csum1 = plsc.cumsum(jnp.where(m & (g == 1), 1, 0))
            plsc.store_scatter(lists, [(ns[0] - 1) + csum0], e, mask=m & (g == 0))
            plsc.store_scatter(lists, [(ARENA - ns[1]) - csum1], e, mask=m & (g == 1))
            return (ns[0] + csum0[15], ns[1] + csum1[15])

        ns = lax.fori_loop(0, CHUNK // 16, pre_body, ns)
    lists[pl.ds(ns[0], 16)] = neg1
    lists[pl.ds(ARENA - ns[1] - 16, 16)] = neg1

    def unpack(j):
        par = j & 1
        toff = par << 7
        for k in range(8):
            e = ring[pl.ds(toff + k * 16, 16)]
            pad = e < 0
            loc = (e >> 14) & 4095
            locb[par, pl.ds(k * 16, 16)] = jnp.where(pad, TRASH_ROW, loc)
            pid = jnp.where(pad, trash_pid, e & 16383) + base_pt

            @pl.when(par == 0)
            def _(pid=pid, k=k):
                pidb0[pl.ds(k * 16, 16)] = pid

            @pl.when(par == 1)
            def _(pid=pid, k=k):
                pidb1[pl.ds(k * 16, 16)] = pid

    def start_gather(j):
        @pl.when((j & 1) == 0)
        def _():
            pltpu.async_copy(hmat.at[pidb0], rows0, sem0)

        @pl.when((j & 1) == 1)
        def _():
            pltpu.async_copy(hmat.at[pidb1], rows1, sem1)

    def drain(j):
        @pl.when((j & 1) == 0)
        def _():
            pltpu.make_async_copy(hmat.at[pl.ds(0, BATCH)], rows0, sem0).wait()
            pltpu.sync_copy(rows0, acc.at[locb.at[0]], add=True)

        @pl.when((j & 1) == 1)
        def _():
            pltpu.make_async_copy(hmat.at[pl.ds(0, BATCH)], rows1, sem1).wait()
            pltpu.sync_copy(rows1, acc.at[locb.at[1]], add=True)

    def fire(j):
        @pl.when(j >= 1)
        def _():
            drain(j - 1)

        unpack(j)
        start_gather(j)

    my_row = s * (D_RANGE // 16)

    def zero_slice():
        for k in range(4):
            pltpu.async_copy(zbuf, acc.at[pl.ds(my_row + k * 64, 64)], sem0)
        for k in range(4):
            pltpu.make_async_copy(zbuf, acc.at[pl.ds(my_row + k * 64, 64)],
                                  sem0).wait()

    zero_slice()
    plsc.subcore_barrier()

    for p in range(N_PASS):
        myq = 2 * p + c
        mrel = ((2 * p) & 31) + c
        g = p >> 4

        def scan_body(i, n, mrel=mrel, g=g):
            if g == 0:
                e = lists[pl.ds(i * 16, 16)]
            else:
                e = lists[pl.ds(ARENA - 16 - i * 16, 16)]
            m = (e >> 26) == mrel
            mi = jnp.where(m, 1, 0)
            pos = ((n - 1) + plsc.cumsum(mi)) & 255
            plsc.store_scatter(ring, [pos], e, mask=m)
            n2 = n + jnp.sum(mi)

            @pl.when((n2 >> 7) > (n >> 7))
            def _():
                fire(n >> 7)

            return n2

        nvec_g = (ns[g] + 15) >> 4
        n = lax.fori_loop(0, nvec_g, scan_body, jnp.int32(0))

        @pl.when((n & 127) > 0)
        def _(n=n):
            for k in range(8):
                plsc.store_scatter(ring, [(n + k * 16 + lane) & 255], neg1)
            fire(n >> 7)

        nb = (n + 127) >> 7

        @pl.when(nb >= 1)
        def _(nb=nb):
            drain(nb - 1)

        plsc.subcore_barrier()
        gbase = myq * D_RANGE + my_row
        pltpu.sync_copy(acc.at[pl.ds(my_row, D_RANGE // 16)],
                        grid_out.at[pl.ds(gbase, D_RANGE // 16)])
        if p < N_PASS - 1:
            zero_slice()
        plsc.subcore_barrier()


def _make_scatter():
    return functools.partial(
        pl.kernel,
        out_type=jax.ShapeDtypeStruct((NX * NY, 128), jnp.float32),
        mesh=plsc.VectorSubcoreMesh(core_axis_name="c", subcore_axis_name="s"),
        compiler_params=pltpu.CompilerParams(needs_layout_passes=False),
        scratch_types=[
            pltpu.VMEM((CHUNK,), jnp.int32),
            pltpu.VMEM((CHUNK,), jnp.int32),
            pltpu.VMEM((ARENA,), jnp.int32),
            pltpu.VMEM((256,), jnp.int32),
            pltpu.VMEM((BATCH, 128), jnp.float32),
            pltpu.VMEM((BATCH, 128), jnp.float32),
            pltpu.VMEM((2, BATCH), jnp.int32),
            pltpu.VMEM((BATCH,), jnp.int32),
            pltpu.VMEM((BATCH,), jnp.int32),
            pltpu.VMEM((64, 128), jnp.float32),
            pltpu.VMEM_SHARED((D_RANGE + 8, 128), jnp.float32),
            pltpu.SemaphoreType.DMA,
            pltpu.SemaphoreType.DMA,
        ],
    )(_scatter_body)


def kernel(x, W, b, gamma, beta, indices):
    xt8 = jnp.zeros((8, N_PAD), jnp.float32).at[:IN_F, :N].set(x.T)

    s1, s2 = _stats(xt8)
    mean_x = s1[:IN_F, 0] / N
    e2 = s2[:IN_F, :IN_F] / N
    mh = W @ mean_x
    mean = mh + b
    eh2 = jnp.einsum("jk,kl,jl->j", W, e2, W) + 2.0 * b * mh + b * b
    var = jnp.maximum(eh2 - mean * mean, 0.0)
    sc = gamma * lax.rsqrt(var + EPS)
    wf8 = jnp.zeros((8, 128), jnp.float32).at[:IN_F, :OUT_F].set((W * sc[:, None]).T)
    bf8 = jnp.zeros((1, 128), jnp.float32).at[0, :OUT_F].set((b - mean) * sc + beta)

    h = _h_compute(xt8, wf8, bf8)

    ix = jnp.full((N_PAD,), NX, jnp.int32).at[:N].set(indices[:, 0])
    iy = jnp.zeros((N_PAD,), jnp.int32).at[:N].set(indices[:, 1])
    zsrc = jnp.zeros((64, 128), jnp.float32)

    grid = _make_scatter()(h, ix, iy, zsrc)
    return grid[:, :OUT_F].reshape(NX, NY, OUT_F)


# speedup vs baseline: 1.5139x; 1.0014x over previous
import functools

import jax
import jax.numpy as jnp
from jax import lax
from jax.experimental import pallas as pl
from jax.experimental.pallas import tpu as pltpu
from jax.experimental.pallas import tpu_sc as plsc

N = 200000
NX = 512
NY = 512
IN_F = 6
OUT_F = 64
EPS = 1e-5

NW = 32
PPW = 12800
N_PAD = 16 * PPW
NVEC = PPW // 16
N_PASS = 32
BATCH = 128
ARENA = PPW + 32

STATS_BLK = 4096
H_BLK = 2048


def _stats_body(xt_ref, s1_ref, s2_ref, a1, a2):
    i = pl.program_id(0)

    @pl.when(i == 0)
    def _():
        a1[...] = jnp.zeros_like(a1)
        a2[...] = jnp.zeros_like(a2)

    xb = xt_ref[...]
    a1[...] += jnp.sum(xb, axis=1, keepdims=True)
    for k in range(IN_F):
        a2[:, k : k + 1] += jnp.sum(xb * xb[k : k + 1, :], axis=1, keepdims=True)

    @pl.when(i == pl.num_programs(0) - 1)
    def _():
        s1_ref[...] = a1[...]
        s2_ref[...] = a2[...]


def _stats(xt):
    return pl.pallas_call(
        _stats_body,
        grid=(N_PAD // STATS_BLK,),
        in_specs=[pl.BlockSpec((8, STATS_BLK), lambda i: (0, i))],
        out_specs=[
            pl.BlockSpec((8, 1), lambda i: (0, 0)),
            pl.BlockSpec((8, 8), lambda i: (0, 0)),
        ],
        out_shape=[
            jax.ShapeDtypeStruct((8, 1), jnp.float32),
            jax.ShapeDtypeStruct((8, 8), jnp.float32),
        ],
        scratch_shapes=[
            pltpu.VMEM((8, 1), jnp.float32),
            pltpu.VMEM((8, 8), jnp.float32),
        ],
    )(xt)


def _h_body(x_ref, w_ref, b_ref, h_ref):
    acc = lax.dot_general(x_ref[...], w_ref[...], (((0,), (0,)), ((), ())),
                          preferred_element_type=jnp.float32)
    h_ref[...] = jnp.maximum(acc + b_ref[...], 0.0)


def _h_compute(x8, wf8, bf8):
    return pl.pallas_call(
        _h_body,
        grid=(N_PAD // H_BLK,),
        in_specs=[
            pl.BlockSpec((8, H_BLK), lambda i: (0, i)),
            pl.BlockSpec((8, 128), lambda i: (0, 0)),
            pl.BlockSpec((1, 128), lambda i: (0, 0)),
        ],
        out_specs=pl.BlockSpec((H_BLK, 128), lambda i: (i, 0)),
        out_shape=jax.ShapeDtypeStruct((N_PAD, 128), jnp.float32),
    )(x8, wf8, bf8)


D_RANGE = 4096
TRASH_ROW = D_RANGE
CHUNK = 3200


def _scatter_body(hmat, ixr, iyr, zsrc, grid_out, ixb, iyb, lists, ring,
                  rows0, rows1, locb, pidb0, pidb1, zbuf, acc, sem0, sem1):
    c = lax.axis_index("c")
    s = lax.axis_index("s")
    base_pt = s * PPW
    lane = lax.iota(jnp.int32, 16)
    neg1 = jnp.full((16,), -1, jnp.int32)
    trash_pid = lane * 399

    pltpu.sync_copy(zsrc, zbuf)

    ns = (jnp.int32(0), jnp.int32(0))
    for ch in range(PPW // CHUNK):
        pltpu.sync_copy(ixr.at[pl.ds(base_pt + ch * CHUNK, CHUNK)], ixb)
        pltpu.sync_copy(iyr.at[pl.ds(base_pt + ch * CHUNK, CHUNK)], iyb)

        def pre_body(i, ns, ch=ch):
            ix = ixb[pl.ds(i * 16, 16)]
            iy = iyb[pl.ds(i * 16, 16)]
            cell = (ix << 9) + iy
            q = cell >> 12
            m = (q < 64) & ((q & 1) == c)
            e = (((q & 31) << 26) | ((cell & 4095) << 14)
                 | (ch * CHUNK + i * 16 + lane))
            g = q >> 5
            csum0 = plsc.cumsum(jnp.where(m & (g == 0), 1, 0))
            csum1 = plsc.cumsum(jnp.where(m & (g == 1), 1, 0))
            plsc.store_scatter(lists, [(ns[0] - 1) + csum0], e, mask=m & (g == 0))
            plsc.store_scatter(lists, [(ARENA - ns[1]) - csum1], e, mask=m & (g == 1))
            return (ns[0] + csum0[15], ns[1] + csum1[15])

        ns = lax.fori_loop(0, CHUNK // 16, pre_body, ns)
    lists[pl.ds(ns[0], 16)] = neg1
    lists[pl.ds(ARENA - ns[1] - 16, 16)] = neg1

    def unpack(j):
        par = j & 1
        toff = par << 7
        for k in range(8):
            e = ring[pl.ds(toff + k * 16, 16)]
            pad = e < 0
            loc = (e >> 14) & 4095
            locb[par, pl.ds(k * 16, 16)] = jnp.where(pad, TRASH_ROW, loc)
            pid = jnp.where(pad, trash_pid, e & 16383) + base_pt

            @pl.when(par == 0)
            def _(pid=pid, k=k):
                pidb0[pl.ds(k * 16, 16)] = pid

            @pl.when(par == 1)
            def _(pid=pid, k=k):
                pidb1[pl.ds(k * 16, 16)] = pid

    def start_gather(j):
        @pl.when((j & 1) == 0)
        def _():
            pltpu.async_copy(hmat.at[pidb0], rows0, sem0)

        @pl.when((j & 1) == 1)
        def _():
            pltpu.async_copy(hmat.at[pidb1], rows1, sem1)

    def drain(j):
        @pl.when((j & 1) == 0)
        def _():
            pltpu.make_async_copy(hmat.at[pl.ds(0, BATCH)], rows0, sem0).wait()
            pltpu.sync_copy(rows0, acc.at[locb.at[0]], add=True)

        @pl.when((j & 1) == 1)
        def _():
            pltpu.make_async_copy(hmat.at[pl.ds(0, BATCH)], rows1, sem1).wait()
            pltpu.sync_copy(rows1, acc.at[locb.at[1]], add=True)

    def fire(j):
        @pl.when(j >= 1)
        def _():
            drain(j - 1)

        unpack(j)
        start_gather(j)

    my_row = s * (D_RANGE // 16)

    def zero_slice():
        for k in range(4):
            pltpu.async_copy(zbuf, acc.at[pl.ds(my_row + k * 64, 64)], sem0)
        for k in range(4):
            pltpu.make_async_copy(zbuf, acc.at[pl.ds(my_row + k * 64, 64)],
                                  sem0).wait()

    zero_slice()
    plsc.subcore_barrier()

    for p in range(N_PASS):
        myq = 2 * p + c
        mrel = ((2 * p) & 31) + c
        g = p >> 4

        def scan_body(i, n, mrel=mrel, g=g):
            if g == 0:
                e = lists[pl.ds(i * 16, 16)]
            else:
                e = lists[pl.ds(ARENA - 16 - i * 16, 16)]
            m = (e >> 26) == mrel
            csum = plsc.cumsum(jnp.where(m, 1, 0))
            pos = ((n - 1) + csum) & 255
            plsc.store_scatter(ring, [pos], e, mask=m)
            n2 = n + csum[15]

            @pl.when((n2 >> 7) > (n >> 7))
            def _():
                fire(n >> 7)

            return n2

        nvec_g = (ns[g] + 15) >> 4
        n = lax.fori_loop(0, nvec_g, scan_body, jnp.int32(0))

        @pl.when((n & 127) > 0)
        def _(n=n):
            for k in range(8):
                plsc.store_scatter(ring, [(n + k * 16 + lane) & 255], neg1)
            fire(n >> 7)

        nb = (n + 127) >> 7

        @pl.when(nb >= 1)
        def _(nb=nb):
            drain(nb - 1)

        plsc.subcore_barrier()
        gbase = myq * D_RANGE + my_row
        pltpu.sync_copy(acc.at[pl.ds(my_row, D_RANGE // 16)],
                        grid_out.at[pl.ds(gbase, D_RANGE // 16)])
        if p < N_PASS - 1:
            zero_slice()
        plsc.subcore_barrier()


def _make_scatter():
    return functools.partial(
        pl.kernel,
        out_type=jax.ShapeDtypeStruct((NX * NY, 128), jnp.float32),
        mesh=plsc.VectorSubcoreMesh(core_axis_name="c", subcore_axis_name="s"),
        compiler_params=pltpu.CompilerParams(needs_layout_passes=False),
        scratch_types=[
            pltpu.VMEM((CHUNK,), jnp.int32),
            pltpu.VMEM((CHUNK,), jnp.int32),
            pltpu.VMEM((ARENA,), jnp.int32),
            pltpu.VMEM((256,), jnp.int32),
            pltpu.VMEM((BATCH, 128), jnp.float32),
            pltpu.VMEM((BATCH, 128), jnp.float32),
            pltpu.VMEM((2, BATCH), jnp.int32),
            pltpu.VMEM((BATCH,), jnp.int32),
            pltpu.VMEM((BATCH,), jnp.int32),
            pltpu.VMEM((64, 128), jnp.float32),
            pltpu.VMEM_SHARED((D_RANGE + 8, 128), jnp.float32),
            pltpu.SemaphoreType.DMA,
            pltpu.SemaphoreType.DMA,
        ],
    )(_scatter_body)


def kernel(x, W, b, gamma, beta, indices):
    xt8 = jnp.zeros((8, N_PAD), jnp.float32).at[:IN_F, :N].set(x.T)

    s1, s2 = _stats(xt8)
    mean_x = s1[:IN_F, 0] / N
    e2 = s2[:IN_F, :IN_F] / N
    mh = W @ mean_x
    mean = mh + b
    eh2 = jnp.einsum("jk,kl,jl->j", W, e2, W) + 2.0 * b * mh + b * b
    var = jnp.maximum(eh2 - mean * mean, 0.0)
    sc = gamma * lax.rsqrt(var + EPS)
    wf8 = jnp.zeros((8, 128), jnp.float32).at[:IN_F, :OUT_F].set((W * sc[:, None]).T)
    bf8 = jnp.zeros((1, 128), jnp.float32).at[0, :OUT_F].set((b - mean) * sc + beta)

    h = _h_compute(xt8, wf8, bf8)

    ix = jnp.full((N_PAD,), NX, jnp.int32).at[:N].set(indices[:, 0])
    iy = jnp.zeros((N_PAD,), jnp.int32).at[:N].set(indices[:, 1])
    zsrc = jnp.zeros((64, 128), jnp.float32)

    grid = _make_scatter()(h, ix, iy, zsrc)
    return grid[:, :OUT_F].reshape(NX, NY, OUT_F)
